# Initial kernel scaffold; baseline (speedup 1.0000x reference)
import jax
import jax.numpy as jnp
from jax.experimental import pallas as pl


def kernel(x, edge_index_1, edge_index_2, num_target, W1, b1, gamma1, beta1, W2, b2):
    raise NotImplementedError("write your pallas kernel here")


# speedup vs baseline: 3.6659x; 3.6659x over previous
import functools

import jax
import jax.numpy as jnp
from jax import lax
from jax.experimental import pallas as pl
from jax.experimental.pallas import tpu as pltpu
from jax.experimental.pallas import tpu_sc as plsc

N = 10000
D_IN = 128
D_HID = 128
D_OUT = 40
NUM_TARGET = 1000

NC = 2
NS = 16
CHUNK = 128


def _ceil_div(a, b):
    return -(-a // b)


def _mm_body(x_ref, w_ref, o_ref):
    o_ref[...] = jnp.dot(x_ref[...], w_ref[...],
                         preferred_element_type=jnp.float32)


def _tc_matmul(x, w, bm):
    m, k = x.shape
    n = w.shape[1]
    grid = m // bm
    return pl.pallas_call(
        _mm_body,
        grid=(grid,),
        in_specs=[
            pl.BlockSpec((bm, k), lambda i: (i, 0)),
            pl.BlockSpec((k, n), lambda i: (0, 0)),
        ],
        out_specs=pl.BlockSpec((bm, n), lambda i: (i, 0)),
        out_shape=jax.ShapeDtypeStruct((m, n), jnp.float32),
    )(x, w)


def _relu_mm_body(p0_ref, p1_ref, c_ref, w_ref, o_ref):
    a = jnp.maximum(p0_ref[...] + p1_ref[...] + c_ref[...], 0.0)
    o_ref[...] = jnp.dot(a, w_ref[...], preferred_element_type=jnp.float32)


def _tc_relu_matmul(parts, cvec, w, bm):
    d = parts.shape[1]
    n = w.shape[1]
    grid = N // bm
    nb = N // bm
    return pl.pallas_call(
        _relu_mm_body,
        grid=(grid,),
        in_specs=[
            pl.BlockSpec((bm, d), lambda i: (i, 0)),
            pl.BlockSpec((bm, d), lambda i, nb=nb: (i + nb, 0)),
            pl.BlockSpec((1, d), lambda i: (0, 0)),
            pl.BlockSpec((d, n), lambda i: (0, 0)),
        ],
        out_specs=pl.BlockSpec((bm, n), lambda i: (i, 0)),
        out_shape=jax.ShapeDtypeStruct((N, n), jnp.float32),
    )(parts, parts, cvec, w)


def _lsm_body(q_ref, b_ref, o_ref, rows_a, rows_b):
    z = (q_ref[0:NUM_TARGET, 0:D_OUT] + q_ref[rows_a:rows_b, 0:D_OUT]
         + b_ref[...])
    m = jnp.max(z, axis=-1, keepdims=True)
    e = jnp.exp(z - m)
    lse = jnp.log(jnp.sum(e, axis=-1, keepdims=True))
    o_ref[...] = z - m - lse


def _tc_logsoftmax(q, b2, rows_per_core):
    body = functools.partial(_lsm_body, rows_a=rows_per_core,
                             rows_b=rows_per_core + NUM_TARGET)
    return pl.pallas_call(
        body,
        grid=(1,),
        in_specs=[
            pl.BlockSpec(q.shape, lambda i: (0, 0)),
            pl.BlockSpec((1, D_OUT), lambda i: (0, 0)),
        ],
        out_specs=pl.BlockSpec((NUM_TARGET, D_OUT), lambda i: (0, 0)),
        out_shape=jax.ShapeDtypeStruct((NUM_TARGET, D_OUT), jnp.float32),
    )(q, b2.reshape(1, D_OUT))


def _sc_agg_body(n_rows, acc_rows, d, chunks_per_tile, out_tiles,
                 h_hbm, src_hbm, dst_hbm, zeros_hbm, out_hbm,
                 idx_s, idx_d, rows, acc, sem):
    c = lax.axis_index("c")
    s = lax.axis_index("s")
    zrows = acc_rows // NS
    orows = n_rows // out_tiles
    pltpu.sync_copy(zeros_hbm, acc.at[pl.ds(s * zrows, zrows)])
    plsc.subcore_barrier()

    per_tile = chunks_per_tile * CHUNK
    base = (c * NS + s) * per_tile

    def step(k, carry):
        off = base + k * CHUNK
        pltpu.sync_copy(src_hbm.at[pl.ds(off, CHUNK)], idx_s)
        pltpu.sync_copy(dst_hbm.at[pl.ds(off, CHUNK)], idx_d)
        pltpu.async_copy(h_hbm.at[idx_s], rows, sem).wait()
        pltpu.sync_copy(rows, acc.at[idx_d], add=True)
        return carry

    lax.fori_loop(0, chunks_per_tile, step, 0)
    plsc.subcore_barrier()

    @pl.when(s < out_tiles)
    def _():
        pltpu.sync_copy(acc.at[pl.ds(s * orows, orows)],
                        out_hbm.at[pl.ds((c * n_rows) + s * orows, orows)])


def _sc_aggregate(h, src, dst, n_rows, acc_rows, chunks_per_tile, out_tiles):
    d = h.shape[1]
    zrows = acc_rows // NS
    zeros = jnp.zeros((zrows, d), jnp.float32)
    mesh = plsc.VectorSubcoreMesh(core_axis_name="c", subcore_axis_name="s")
    body = functools.partial(_sc_agg_body, n_rows, acc_rows, d,
                             chunks_per_tile, out_tiles)
    return pl.kernel(
        body,
        out_type=jax.ShapeDtypeStruct((NC * n_rows, d), jnp.float32),
        mesh=mesh,
        scratch_types=[
            pltpu.VMEM((CHUNK,), jnp.int32),
            pltpu.VMEM((CHUNK,), jnp.int32),
            pltpu.VMEM((CHUNK, d), jnp.float32),
            pltpu.VMEM_SHARED((acc_rows, d), jnp.float32),
            pltpu.SemaphoreType.DMA,
        ],
    )(h, src, dst, zeros)


def _pad_edges(src, dst, e, e_pad, junk):
    pad = e_pad - e
    srcp = jnp.concatenate([src, jnp.zeros((pad,), jnp.int32)])
    dstp = jnp.concatenate([dst, jnp.full((pad,), junk, jnp.int32)])
    return srcp, dstp


def kernel(x, edge_index_1, edge_index_2, num_target,
           W1, b1, gamma1, beta1, W2, b2):
    eps = 1e-5
    scale = gamma1 / jnp.sqrt(1.0 + eps)
    w1s = W1 * scale[None, :]
    cvec = (b1 * scale + beta1).reshape(1, D_HID)

    e1 = edge_index_1.shape[1]
    e2 = edge_index_2.shape[1]
    ch1 = _ceil_div(e1, NC * NS * CHUNK)
    ch2 = _ceil_div(e2, NC * NS * CHUNK)
    e1p = NC * NS * ch1 * CHUNK
    e2p = NC * NS * ch2 * CHUNK

    acc1_rows = _ceil_div(N + 1, NS * 8) * NS * 8
    src1, dst1 = _pad_edges(edge_index_1[0], edge_index_1[1], e1, e1p, N)

    acc2_rows = _ceil_div(NUM_TARGET + 1, NS * 8) * NS * 8
    dst2 = jnp.where(edge_index_2[1] < NUM_TARGET, edge_index_2[1],
                     NUM_TARGET)
    src2, dst2 = _pad_edges(edge_index_2[0], dst2, e2, e2p, NUM_TARGET)

    w2p = jnp.pad(W2, ((0, 0), (0, 128 - D_OUT)))

    h = _tc_matmul(x, w1s, bm=1000)
    parts1 = _sc_aggregate(h, src1, dst1, N, acc1_rows, ch1, 10)
    h2 = _tc_relu_matmul(parts1, cvec, w2p, bm=1000)
    parts2 = _sc_aggregate(h2, src2, dst2, acc2_rows, acc2_rows, ch2,
                           NS)
    return _tc_logsoftmax(parts2, b2, acc2_rows)


# speedup vs baseline: 3.8767x; 1.0575x over previous
import functools

import jax
import jax.numpy as jnp
from jax import lax
from jax.experimental import pallas as pl
from jax.experimental.pallas import tpu as pltpu
from jax.experimental.pallas import tpu_sc as plsc

N = 10000
D_IN = 128
D_HID = 128
D_OUT = 40
NUM_TARGET = 1000

NC = 2
NS = 16
CHUNK = 128


def _ceil_div(a, b):
    return -(-a // b)


def _mm_body(x_ref, w_ref, o_ref):
    o_ref[...] = jnp.dot(x_ref[...], w_ref[...],
                         preferred_element_type=jnp.float32)


def _tc_matmul(x, w, bm):
    m, k = x.shape
    n = w.shape[1]
    grid = m // bm
    return pl.pallas_call(
        _mm_body,
        grid=(grid,),
        in_specs=[
            pl.BlockSpec((bm, k), lambda i: (i, 0)),
            pl.BlockSpec((k, n), lambda i: (0, 0)),
        ],
        out_specs=pl.BlockSpec((bm, n), lambda i: (i, 0)),
        out_shape=jax.ShapeDtypeStruct((m, n), jnp.float32),
    )(x, w)


def _relu_mm_body(p0_ref, p1_ref, c_ref, w_ref, o_ref):
    a = jnp.maximum(p0_ref[...] + p1_ref[...] + c_ref[...], 0.0)
    o_ref[...] = jnp.dot(a, w_ref[...], preferred_element_type=jnp.float32)


def _tc_relu_matmul(parts, cvec, w, bm):
    d = parts.shape[1]
    n = w.shape[1]
    grid = N // bm
    nb = N // bm
    return pl.pallas_call(
        _relu_mm_body,
        grid=(grid,),
        in_specs=[
            pl.BlockSpec((bm, d), lambda i: (i, 0)),
            pl.BlockSpec((bm, d), lambda i, nb=nb: (i + nb, 0)),
            pl.BlockSpec((1, d), lambda i: (0, 0)),
            pl.BlockSpec((d, n), lambda i: (0, 0)),
        ],
        out_specs=pl.BlockSpec((bm, n), lambda i: (i, 0)),
        out_shape=jax.ShapeDtypeStruct((N, n), jnp.float32),
    )(parts, parts, cvec, w)


def _lsm_body(q_ref, b_ref, o_ref, rows_a, rows_b):
    z = (q_ref[0:NUM_TARGET, 0:D_OUT] + q_ref[rows_a:rows_b, 0:D_OUT]
         + b_ref[...])
    m = jnp.max(z, axis=-1, keepdims=True)
    e = jnp.exp(z - m)
    lse = jnp.log(jnp.sum(e, axis=-1, keepdims=True))
    o_ref[...] = z - m - lse


def _tc_logsoftmax(q, b2, rows_per_core):
    body = functools.partial(_lsm_body, rows_a=rows_per_core,
                             rows_b=rows_per_core + NUM_TARGET)
    return pl.pallas_call(
        body,
        grid=(1,),
        in_specs=[
            pl.BlockSpec(q.shape, lambda i: (0, 0)),
            pl.BlockSpec((1, D_OUT), lambda i: (0, 0)),
        ],
        out_specs=pl.BlockSpec((NUM_TARGET, D_OUT), lambda i: (0, 0)),
        out_shape=jax.ShapeDtypeStruct((NUM_TARGET, D_OUT), jnp.float32),
    )(q, b2.reshape(1, D_OUT))


def _sc_agg_body(n_rows, acc_rows, d, chunks_per_tile, out_tiles,
                 h_hbm, src_hbm, dst_hbm, zeros_hbm, out_hbm,
                 idx_s0, idx_s1, idx_d0, idx_d1, rows0, rows1,
                 acc, sem0, sem1):
    c = lax.axis_index("c")
    s = lax.axis_index("s")
    zrows = acc_rows // NS
    orows = n_rows // out_tiles
    pltpu.sync_copy(zeros_hbm, acc.at[pl.ds(s * zrows, zrows)])
    plsc.subcore_barrier()

    per_tile = chunks_per_tile * CHUNK
    base = (c * NS + s) * per_tile
    idx_s = (idx_s0, idx_s1)
    idx_d = (idx_d0, idx_d1)
    rows = (rows0, rows1)
    sems = (sem0, sem1)

    def load_and_fire(k, b):
        off = base + k * CHUNK
        pltpu.sync_copy(src_hbm.at[pl.ds(off, CHUNK)], idx_s[b])
        pltpu.sync_copy(dst_hbm.at[pl.ds(off, CHUNK)], idx_d[b])
        pltpu.async_copy(h_hbm.at[idx_s[b]], rows[b], sems[b])

    load_and_fire(0, 0)

    @pl.loop(0, chunks_per_tile, step=2)
    def _(k):
        for b in range(2):
            kk = k + b

            @pl.when(kk + 1 < chunks_per_tile)
            def _():
                load_and_fire(kk + 1, 1 - b)

            pltpu.make_async_copy(h_hbm.at[idx_s[b]], rows[b],
                                  sems[b]).wait()
            pltpu.sync_copy(rows[b], acc.at[idx_d[b]], add=True)

    plsc.subcore_barrier()

    @pl.when(s < out_tiles)
    def _():
        pltpu.sync_copy(acc.at[pl.ds(s * orows, orows)],
                        out_hbm.at[pl.ds((c * n_rows) + s * orows, orows)])


def _sc_aggregate(h, src, dst, n_rows, acc_rows, chunks_per_tile, out_tiles):
    d = h.shape[1]
    zrows = acc_rows // NS
    zeros = jnp.zeros((zrows, d), jnp.float32)
    mesh = plsc.VectorSubcoreMesh(core_axis_name="c", subcore_axis_name="s")
    body = functools.partial(_sc_agg_body, n_rows, acc_rows, d,
                             chunks_per_tile, out_tiles)
    return pl.kernel(
        body,
        out_type=jax.ShapeDtypeStruct((NC * n_rows, d), jnp.float32),
        mesh=mesh,
        scratch_types=[
            pltpu.VMEM((CHUNK,), jnp.int32),
            pltpu.VMEM((CHUNK,), jnp.int32),
            pltpu.VMEM((CHUNK,), jnp.int32),
            pltpu.VMEM((CHUNK,), jnp.int32),
            pltpu.VMEM((CHUNK, d), jnp.float32),
            pltpu.VMEM((CHUNK, d), jnp.float32),
            pltpu.VMEM_SHARED((acc_rows, d), jnp.float32),
            pltpu.SemaphoreType.DMA,
            pltpu.SemaphoreType.DMA,
        ],
    )(h, src, dst, zeros)


def _pad_edges(src, dst, e, e_pad, junk_lo, junk_n):
    pad = e_pad - e
    junk = junk_lo + (jnp.arange(pad, dtype=jnp.int32) % junk_n)
    srcp = jnp.concatenate([src, jnp.zeros((pad,), jnp.int32)])
    dstp = jnp.concatenate([dst, junk])
    return srcp, dstp


def kernel(x, edge_index_1, edge_index_2, num_target,
           W1, b1, gamma1, beta1, W2, b2):
    eps = 1e-5
    scale = gamma1 / jnp.sqrt(1.0 + eps)
    w1s = W1 * scale[None, :]
    cvec = (b1 * scale + beta1).reshape(1, D_HID)

    e1 = edge_index_1.shape[1]
    e2 = edge_index_2.shape[1]
    ch1 = (_ceil_div(e1, NC * NS * CHUNK) + 1) // 2 * 2
    ch2 = (_ceil_div(e2, NC * NS * CHUNK) + 1) // 2 * 2
    e1p = NC * NS * ch1 * CHUNK
    e2p = NC * NS * ch2 * CHUNK

    acc1_rows = _ceil_div(N + 1, NS * 8) * NS * 8
    src1, dst1 = _pad_edges(edge_index_1[0], edge_index_1[1], e1, e1p,
                            N, acc1_rows - N)

    acc2_rows = _ceil_div(NUM_TARGET + 1, NS * 8) * NS * 8
    d2 = edge_index_2[1]
    dst2 = jnp.where(d2 < NUM_TARGET, d2,
                     NUM_TARGET + lax.rem(d2, acc2_rows - NUM_TARGET))
    src2, dst2 = _pad_edges(edge_index_2[0], dst2, e2, e2p,
                            NUM_TARGET, acc2_rows - NUM_TARGET)

    w2p = jnp.pad(W2, ((0, 0), (0, 128 - D_OUT)))

    h = _tc_matmul(x, w1s, bm=1000)
    parts1 = _sc_aggregate(h, src1, dst1, N, acc1_rows, ch1, 10)
    h2 = _tc_relu_matmul(parts1, cvec, w2p, bm=1000)
    parts2 = _sc_aggregate(h2, src2, dst2, acc2_rows, acc2_rows, ch2,
                           NS)
    return _tc_logsoftmax(parts2, b2, acc2_rows)
